# Initial kernel scaffold; baseline (speedup 1.0000x reference)
import functools

import jax
import jax.numpy as jnp
from jax import lax
from jax.experimental import pallas as pl
from jax.experimental.pallas import tpu as pltpu
from jax.experimental.pallas import tpu_sc as plsc

B = 4096
HIST = 50
D = 64
U = 128

NC = 2
NS = 16
NW = NC * NS
ROWS_PER_W = B // NW
PAIR = 2
IDS_PER_GATHER = PAIR * HIST
NCHUNK = ROWS_PER_W // PAIR


def _sc_pooled_body(table_hbm, ids_hbm, out_hbm, idx_v, rows_v, pooled_v, sem):
  cid = lax.axis_index("c")
  sid = lax.axis_index("s")
  wid = sid * NC + cid

  pltpu.sync_copy(ids_hbm.at[wid], idx_v)

  def chunk_body(j, carry):
    pltpu.make_async_copy(table_hbm.at[idx_v.at[j]], rows_v, sem).start()
    pltpu.make_async_copy(table_hbm.at[idx_v.at[j]], rows_v, sem).wait()
    for r in range(PAIR):
      zero = jnp.zeros((16,), jnp.float32)

      def l_body(l, accs):
        row = r * HIST + l
        out = []
        for v in range(4):
          x = rows_v[row, pl.ds(v * 16, 16)]
          out.append(accs[v] + x * x)
        return tuple(out)

      accs = lax.fori_loop(0, HIST, l_body, (zero, zero, zero, zero))
      orow = j * PAIR + r
      for v in range(4):
        pooled_v[orow, pl.ds(v * 16, 16)] = accs[v]
    return carry

  lax.fori_loop(0, NCHUNK, chunk_body, 0)

  pltpu.sync_copy(pooled_v, out_hbm.at[pl.ds(wid * ROWS_PER_W, ROWS_PER_W)])


@jax.jit
def _sc_pooled(embedding, ids3):
  mesh = plsc.VectorSubcoreMesh(core_axis_name="c", subcore_axis_name="s")
  return pl.kernel(
      _sc_pooled_body,
      out_type=jax.ShapeDtypeStruct((B, D), jnp.float32),
      mesh=mesh,
      scratch_types=[
          pltpu.VMEM((NCHUNK, IDS_PER_GATHER), jnp.int32),
          pltpu.VMEM((IDS_PER_GATHER, D), jnp.float32),
          pltpu.VMEM((ROWS_PER_W, D), jnp.float32),
          pltpu.SemaphoreType.DMA,
      ],
  )(embedding, ids3)


def _mm_body(p_ref, k_ref, o_ref):
  o_ref[...] = jnp.dot(p_ref[...], k_ref[...],
                       preferred_element_type=jnp.float32)


@jax.jit
def _dense(pooled, w):
  return pl.pallas_call(
      _mm_body,
      out_shape=jax.ShapeDtypeStruct((B, U), jnp.float32),
  )(pooled, w)


def kernel(input_ids, embedding, kernel):
  ids3 = input_ids.reshape(NW, NCHUNK, IDS_PER_GATHER)
  pooled = _sc_pooled(embedding, ids3)
  return _dense(pooled, kernel)


# speedup vs baseline: 6.4526x; 6.4526x over previous
import functools

import jax
import jax.numpy as jnp
from jax import lax
from jax.experimental import pallas as pl
from jax.experimental.pallas import tpu as pltpu
from jax.experimental.pallas import tpu_sc as plsc

B = 4096
HIST = 50
D = 64
U = 128

NC = 2
NS = 16
NW = NC * NS
ROWS_PER_W = B // NW
PAIR = 2
IDS_PER_GATHER = PAIR * HIST
NCHUNK = ROWS_PER_W // PAIR


def _sc_pooled_body(table_hbm, ids_hbm, out_hbm, idx_v, rows_v, pooled_v, sem):
  cid = lax.axis_index("c")
  sid = lax.axis_index("s")
  wid = sid * NC + cid

  pltpu.sync_copy(ids_hbm.at[wid], idx_v)

  def chunk_body(j, carry):
    pltpu.make_async_copy(table_hbm.at[idx_v.at[j]], rows_v, sem).start()
    pltpu.make_async_copy(table_hbm.at[idx_v.at[j]], rows_v, sem).wait()
    for r in range(PAIR):
      zero = jnp.zeros((16,), jnp.float32)

      def l_body(l, accs):
        row = r * HIST + l
        out = []
        for v in range(4):
          x = rows_v[row, pl.ds(v * 16, 16)]
          out.append(accs[v] + x * x)
        return tuple(out)

      accs = lax.fori_loop(0, HIST, l_body, (zero, zero, zero, zero))
      orow = j * PAIR + r
      for v in range(4):
        pooled_v[orow, pl.ds(v * 16, 16)] = accs[v]
    return carry

  lax.fori_loop(0, NCHUNK, chunk_body, 0)

  pltpu.sync_copy(pooled_v, out_hbm.at[pl.ds(wid * ROWS_PER_W, ROWS_PER_W)])


@jax.jit
def _sc_pooled(embedding, ids3):
  mesh = plsc.VectorSubcoreMesh(core_axis_name="c", subcore_axis_name="s")
  return pl.kernel(
      _sc_pooled_body,
      out_type=jax.ShapeDtypeStruct((B, D), jnp.float32),
      mesh=mesh,
      compiler_params=pltpu.CompilerParams(use_tc_tiling_on_sc=False),
      scratch_types=[
          pltpu.VMEM((NCHUNK, IDS_PER_GATHER), jnp.int32),
          pltpu.VMEM((IDS_PER_GATHER, D), jnp.float32),
          pltpu.VMEM((ROWS_PER_W, D), jnp.float32),
          pltpu.SemaphoreType.DMA,
      ],
  )(embedding, ids3)


def _mm_body(p_ref, k_ref, o_ref):
  o_ref[...] = jnp.dot(p_ref[...], k_ref[...],
                       preferred_element_type=jnp.float32)


@jax.jit
def _dense(pooled, w):
  return pl.pallas_call(
      _mm_body,
      out_shape=jax.ShapeDtypeStruct((B, U), jnp.float32),
  )(pooled, w)


def kernel(input_ids, embedding, kernel):
  ids3 = input_ids.reshape(NW, NCHUNK, IDS_PER_GATHER)
  pooled = _sc_pooled(embedding, ids3)
  return _dense(pooled, kernel)


# speedup vs baseline: 8.4052x; 1.3026x over previous
import functools

import jax
import jax.numpy as jnp
from jax import lax
from jax.experimental import pallas as pl
from jax.experimental.pallas import tpu as pltpu
from jax.experimental.pallas import tpu_sc as plsc

B = 4096
HIST = 50
D = 64
U = 128

NC = 2
NS = 16
NW = NC * NS
ROWS_PER_W = B // NW
PAIR = 2
IDS_PER_GATHER = PAIR * HIST
NCHUNK = ROWS_PER_W // PAIR


NBUF = 2


def _sc_pooled_body(table_hbm, ids_hbm, out_hbm, idx_v, rows_v, pooled_v,
                    sem0, sem1):
  sems = (sem0, sem1)
  cid = lax.axis_index("c")
  sid = lax.axis_index("s")
  wid = sid * NC + cid

  pltpu.sync_copy(ids_hbm.at[wid], idx_v)

  for b in range(NBUF):
    pltpu.make_async_copy(
        table_hbm.at[idx_v.at[b]], rows_v.at[b], sems[b]).start()

  def group_body(g, carry):
    for b in range(NBUF):
      j = g * NBUF + b
      pltpu.make_async_copy(
          table_hbm.at[idx_v.at[j]], rows_v.at[b], sems[b]).wait()
      for r in range(PAIR):
        acc = [jnp.zeros((16,), jnp.float32) for _ in range(4)]
        for l in range(HIST):
          row = r * HIST + l
          for v in range(4):
            x = rows_v[b, row, pl.ds(v * 16, 16)]
            acc[v] = acc[v] + x * x
        orow = j * PAIR + r
        for v in range(4):
          pooled_v[orow, pl.ds(v * 16, 16)] = acc[v]

      @pl.when(j + NBUF < NCHUNK)
      def _refill():
        pltpu.make_async_copy(
            table_hbm.at[idx_v.at[j + NBUF]], rows_v.at[b], sems[b]).start()
    return carry

  lax.fori_loop(0, NCHUNK // NBUF, group_body, 0)

  pltpu.sync_copy(pooled_v, out_hbm.at[pl.ds(wid * ROWS_PER_W, ROWS_PER_W)])


@jax.jit
def _sc_pooled(embedding, ids3):
  mesh = plsc.VectorSubcoreMesh(core_axis_name="c", subcore_axis_name="s")
  return pl.kernel(
      _sc_pooled_body,
      out_type=jax.ShapeDtypeStruct((B, D), jnp.float32),
      mesh=mesh,
      compiler_params=pltpu.CompilerParams(use_tc_tiling_on_sc=False),
      scratch_types=[
          pltpu.VMEM((NCHUNK, IDS_PER_GATHER), jnp.int32),
          pltpu.VMEM((NBUF, IDS_PER_GATHER, D), jnp.float32),
          pltpu.VMEM((ROWS_PER_W, D), jnp.float32),
          pltpu.SemaphoreType.DMA,
          pltpu.SemaphoreType.DMA,
      ],
  )(embedding, ids3)


def _mm_body(p_ref, k_ref, o_ref):
  o_ref[...] = jnp.dot(p_ref[...], k_ref[...],
                       preferred_element_type=jnp.float32)


@jax.jit
def _dense(pooled, w):
  return pl.pallas_call(
      _mm_body,
      out_shape=jax.ShapeDtypeStruct((B, U), jnp.float32),
  )(pooled, w)


def kernel(input_ids, embedding, kernel):
  ids3 = input_ids.reshape(NW, NCHUNK, IDS_PER_GATHER)
  pooled = _sc_pooled(embedding, ids3)
  return _dense(pooled, kernel)


# speedup vs baseline: 8.9548x; 1.0654x over previous
import jax
import jax.numpy as jnp
from jax import lax
from jax.experimental import pallas as pl
from jax.experimental.pallas import tpu as pltpu
from jax.experimental.pallas import tpu_sc as plsc

B = 4096
HIST = 50
D = 64
U = 128

NC = 2
NS = 16
NW = NC * NS
ROWS_PER_W = B // NW
NBUF = 4


def _sc_pooled_body(table_hbm, ids_hbm, out_hbm, idx_v, rows_v, pooled_v,
                    *sems):
  cid = lax.axis_index("c")
  sid = lax.axis_index("s")
  wid = sid * NC + cid
  base = wid * ROWS_PER_W

  pltpu.sync_copy(ids_hbm.at[pl.ds(base, ROWS_PER_W)], idx_v)

  for b in range(NBUF):
    pltpu.make_async_copy(
        table_hbm.at[idx_v.at[b]], rows_v.at[b], sems[b]).start()

  def group_body(g, carry):
    for b in range(NBUF):
      j = g * NBUF + b
      pltpu.make_async_copy(
          table_hbm.at[idx_v.at[j]], rows_v.at[b], sems[b]).wait()
      acc = [jnp.zeros((16,), jnp.float32) for _ in range(4)]
      for l in range(HIST):
        for v in range(4):
          x = rows_v[b, l, pl.ds(v * 16, 16)]
          acc[v] = acc[v] + x * x
      for v in range(4):
        pooled_v[j, pl.ds(v * 16, 16)] = acc[v]

      @pl.when(j + NBUF < ROWS_PER_W)
      def _refill():
        pltpu.make_async_copy(
            table_hbm.at[idx_v.at[j + NBUF]], rows_v.at[b], sems[b]).start()
    return carry

  lax.fori_loop(0, ROWS_PER_W // NBUF, group_body, 0)

  pltpu.sync_copy(pooled_v, out_hbm.at[pl.ds(base, ROWS_PER_W)])


@jax.jit
def _sc_pooled(embedding, ids):
  mesh = plsc.VectorSubcoreMesh(core_axis_name="c", subcore_axis_name="s")
  return pl.kernel(
      _sc_pooled_body,
      out_type=jax.ShapeDtypeStruct((B, D), jnp.float32),
      mesh=mesh,
      compiler_params=pltpu.CompilerParams(use_tc_tiling_on_sc=False),
      scratch_types=[
          pltpu.VMEM((ROWS_PER_W, HIST), jnp.int32),
          pltpu.VMEM((NBUF, HIST, D), jnp.float32),
          pltpu.VMEM((ROWS_PER_W, D), jnp.float32),
      ] + [pltpu.SemaphoreType.DMA] * NBUF,
  )(embedding, ids)


def _mm_body(p_ref, k_ref, o_ref):
  o_ref[...] = jnp.dot(p_ref[...], k_ref[...],
                       preferred_element_type=jnp.float32)


@jax.jit
def _dense(pooled, w):
  return pl.pallas_call(
      _mm_body,
      out_shape=jax.ShapeDtypeStruct((B, U), jnp.float32),
  )(pooled, w)


def kernel(input_ids, embedding, kernel):
  pooled = _sc_pooled(embedding, input_ids)
  return _dense(pooled, kernel)
